# baseline (speedup 1.0000x reference)
import functools

import jax
import jax.numpy as jnp
from jax import lax
from jax.experimental import pallas as pl
from jax.experimental.pallas import tpu as pltpu
from jax.experimental.pallas import tpu_sc as plsc

B = 16384
D = 32
NU = 1000000
NC = 2
NS = 16
L = 16
NW = NC * NS
BPW = B // NW
CK = 128
NCK = BPW // CK
FPW = BPW * D


@functools.lru_cache(maxsize=None)
def _make_sc_gather():
    mesh = plsc.VectorSubcoreMesh(core_axis_name="c", subcore_axis_name="s")

    @functools.partial(
        pl.kernel,
        mesh=mesh,
        compiler_params=pltpu.CompilerParams(needs_layout_passes=False),
        out_type=[
            jax.ShapeDtypeStruct((B * D,), jnp.float32),
            jax.ShapeDtypeStruct((B * D,), jnp.float32),
            jax.ShapeDtypeStruct((B,), jnp.float32),
        ],
        scratch_types=[
            pltpu.VMEM((BPW,), jnp.int32),
            pltpu.VMEM((BPW,), jnp.int32),
            pltpu.VMEM((FPW,), jnp.int32),
            pltpu.VMEM((FPW,), jnp.int32),
            pltpu.VMEM((FPW,), jnp.float32),
            pltpu.VMEM((FPW,), jnp.float32),
            pltpu.VMEM((BPW,), jnp.float32),
            pltpu.VMEM((BPW,), jnp.float32),
            pltpu.VMEM((BPW,), jnp.float32),
            pltpu.SemaphoreType.DMA,
        ],
    )
    def sc_gather(uid_hbm, iid_hbm, uembt_hbm, iembt_hbm, ulin_hbm,
                  ilin_hbm, ue_out, ie_out, lin_out,
                  uidx_v, iidx_v, uix_all, iix_all, ue_f, ie_f, ul_f, il_f,
                  lin_f, sem):
        wid = lax.axis_index("s") * NC + lax.axis_index("c")
        base = wid * BPW
        pltpu.sync_copy(uid_hbm.at[pl.ds(base, BPW)], uidx_v)
        pltpu.sync_copy(iid_hbm.at[pl.ds(base, BPW)], iidx_v)

        def build(g, _):
            uv = uidx_v[pl.ds(g * L, L)]
            iv = iidx_v[pl.ds(g * L, L)]
            for j in range(D):
                o = j * BPW + g * L
                uix_all[pl.ds(o, L)] = uv + j * NU
                iix_all[pl.ds(o, L)] = iv + j * NU
            return 0

        lax.fori_loop(0, BPW // L, build, 0)

        for c in range(NCK):
            sl = pl.ds(c * CK, CK)
            pltpu.async_copy(ulin_hbm.at[uidx_v.at[sl]], ul_f.at[sl], sem)
            pltpu.async_copy(ilin_hbm.at[iidx_v.at[sl]], il_f.at[sl], sem)

        def fire(j, _):
            for c in range(NCK):
                o = pl.ds(j * BPW + c * CK, CK)
                pltpu.async_copy(uembt_hbm.at[uix_all.at[o]], ue_f.at[o],
                                 sem)
                pltpu.async_copy(iembt_hbm.at[iix_all.at[o]], ie_f.at[o],
                                 sem)
            return 0

        lax.fori_loop(0, D, fire, 0)

        pltpu.make_async_copy(ue_out.at[pl.ds(0, FPW)], ue_f, sem).wait()
        pltpu.make_async_copy(ie_out.at[pl.ds(0, FPW)], ie_f, sem).wait()
        pltpu.make_async_copy(lin_out.at[pl.ds(0, BPW)], ul_f, sem).wait()
        pltpu.make_async_copy(lin_out.at[pl.ds(0, BPW)], il_f, sem).wait()

        def lsum(k, _):
            sl = pl.ds(k * L, L)
            lin_f[sl] = ul_f[sl] + il_f[sl]
            return 0

        lax.fori_loop(0, BPW // L, lsum, 0)

        pltpu.sync_copy(ue_f, ue_out.at[pl.ds(wid * FPW, FPW)])
        pltpu.sync_copy(ie_f, ie_out.at[pl.ds(wid * FPW, FPW)])
        pltpu.sync_copy(lin_f, lin_out.at[pl.ds(base, BPW)])

    return sc_gather


def _sc_gather(*args):
    return _make_sc_gather()(*args)


def _dense_body(ue_ref, ie_ref, lin_ref, w0u_ref, w0i_ref, b0_ref,
                w1_ref, b1_ref, wout_ref, ones_ref, c_ref, out_ref):
    ue = ue_ref[...]
    ie = ie_ref[...]
    dn = (((1,), (0,)), ((), ()))
    h0 = lax.dot_general(w0u_ref[...], ue, dn,
                         preferred_element_type=jnp.float32)
    h0 = h0 + lax.dot_general(w0i_ref[...], ie, dn,
                              preferred_element_type=jnp.float32)
    h0 = jnp.maximum(h0 + b0_ref[...], 0.0)
    h1 = lax.dot_general(w1_ref[...], h0, dn,
                         preferred_element_type=jnp.float32)
    h1 = jnp.maximum(h1 + b1_ref[...], 0.0)
    dnn = lax.dot_general(wout_ref[...], h1, dn,
                          preferred_element_type=jnp.float32)
    fm = lax.dot_general(ones_ref[...], ue * ie, dn,
                         preferred_element_type=jnp.float32)
    logit = lin_ref[0] + fm + dnn + c_ref[0]
    out_ref[...] = (1.0 / (1.0 + jnp.exp(-logit)))[None]


def _dense(ue, ie, lin, w0u, w0i, b0c, w1, b1c, wout, ones_blk, c):
    SB = 4
    RD = SB * D
    grid = (NW // SB,)
    return pl.pallas_call(
        _dense_body,
        grid=grid,
        in_specs=[
            pl.BlockSpec((RD, BPW), lambda i: (i, 0)),
            pl.BlockSpec((RD, BPW), lambda i: (i, 0)),
            pl.BlockSpec((1, SB, BPW), lambda i: (i, 0, 0)),
            pl.BlockSpec((RD, RD), lambda i: (0, 0)),
            pl.BlockSpec((RD, RD), lambda i: (0, 0)),
            pl.BlockSpec((RD, 1), lambda i: (0, 0)),
            pl.BlockSpec((RD, RD), lambda i: (0, 0)),
            pl.BlockSpec((RD, 1), lambda i: (0, 0)),
            pl.BlockSpec((SB, RD), lambda i: (0, 0)),
            pl.BlockSpec((SB, RD), lambda i: (0, 0)),
            pl.BlockSpec(memory_space=pltpu.SMEM),
        ],
        out_specs=pl.BlockSpec((1, SB, BPW), lambda i: (i, 0, 0)),
        out_shape=jax.ShapeDtypeStruct((NW // SB, SB, BPW), jnp.float32),
    )(ue, ie, lin, w0u, w0i, b0c, w1, b1c, wout, ones_blk, c)


def kernel(user_ids, item_ids, user_embedding, item_embedding, user_linear,
           item_linear, W0, b0, g0, beta0, W1, b1, g1, beta1, W_out, b_out,
           bias):
    eps = 1e-5
    s = 1.0 / jnp.sqrt(1.0 + eps)
    s0 = g0 * s
    s1 = g1 * s
    W0f = W0 * s0[None, :]
    b0f = b0 * s0 + beta0
    W1f = W1 * s1[None, :]
    b1f = b1 * s1 + beta1
    c = (b_out + bias).reshape((1,))

    eye = jnp.eye(4, dtype=jnp.float32)
    w0u_blk = jnp.kron(eye, jnp.transpose(W0f[:D]))
    w0i_blk = jnp.kron(eye, jnp.transpose(W0f[D:]))
    w1_blk = jnp.kron(eye, jnp.transpose(W1f))
    wout_blk = jnp.kron(eye, jnp.transpose(W_out))
    ones_blk = jnp.kron(eye, jnp.ones((1, D), jnp.float32))
    b0c = jnp.tile(b0f, 4).reshape((4 * D, 1))
    b1c = jnp.tile(b1f, 4).reshape((4 * D, 1))

    uembt = jnp.transpose(user_embedding).reshape((NU * D,))
    iembt = jnp.transpose(item_embedding).reshape((NU * D,))
    ulin = user_linear.reshape((NU,))
    ilin = item_linear.reshape((NU,))

    uet, iet, lin = _sc_gather(
        user_ids.astype(jnp.int32), item_ids.astype(jnp.int32),
        uembt, iembt, ulin, ilin)
    ue2 = uet.reshape((NW * D, BPW))
    ie2 = iet.reshape((NW * D, BPW))
    lin2 = lin.reshape((NW // 4, 4, BPW))

    out = _dense(ue2, ie2, lin2, w0u_blk, w0i_blk, b0c, w1_blk, b1c,
                 wout_blk, ones_blk, c)
    return out.reshape((B,))


# speedup vs baseline: 5.6568x; 5.6568x over previous
import functools

import jax
import jax.numpy as jnp
from jax import lax
from jax.experimental import pallas as pl
from jax.experimental.pallas import tpu as pltpu
from jax.experimental.pallas import tpu_sc as plsc

B = 16384
D = 32
NU = 1000000
PK = 4
NC = 2
NS = 16
L = 16
NW = NC * NS
BPW = B // NW
CK = 128
FPW = BPW * D
BP = B // PK
CH = 256
NCH = BPW // CH


@functools.lru_cache(maxsize=None)
def _make_sc_gather():
    mesh = plsc.VectorSubcoreMesh(core_axis_name="c", subcore_axis_name="s")

    @functools.partial(
        pl.kernel,
        mesh=mesh,
        compiler_params=pltpu.CompilerParams(needs_layout_passes=False),
        out_type=[
            jax.ShapeDtypeStruct((B * D,), jnp.float32),
            jax.ShapeDtypeStruct((B * D,), jnp.float32),
            jax.ShapeDtypeStruct((B,), jnp.float32),
        ],
        scratch_types=[
            pltpu.VMEM((BPW,), jnp.int32),
            pltpu.VMEM((BPW,), jnp.int32),
            pltpu.VMEM((BPW,), jnp.int32),
            pltpu.VMEM((BPW,), jnp.int32),
            pltpu.VMEM((CH, PK * D), jnp.float32),
            pltpu.VMEM((CH, PK * D), jnp.float32),
            pltpu.VMEM((FPW,), jnp.float32),
            pltpu.VMEM((FPW,), jnp.float32),
            pltpu.VMEM((BPW,), jnp.float32),
            pltpu.VMEM((BPW,), jnp.float32),
            pltpu.VMEM((BPW,), jnp.float32),
            pltpu.SemaphoreType.DMA,
        ],
    )
    def sc_gather(uid_hbm, iid_hbm, uemb_hbm, iemb_hbm, ulin_hbm, ilin_hbm,
                  ue_out, ie_out, lin_out,
                  uidx_v, iidx_v, usr_v, isr_v, ue_g, ie_g, ue_f, ie_f,
                  ul_f, il_f, lin_f, sem):
        wid = lax.axis_index("s") * NC + lax.axis_index("c")
        base = wid * BPW
        pltpu.sync_copy(uid_hbm.at[pl.ds(base, BPW)], uidx_v)
        pltpu.sync_copy(iid_hbm.at[pl.ds(base, BPW)], iidx_v)

        for c in range(BPW // CK):
            sl = pl.ds(c * CK, CK)
            pltpu.async_copy(ulin_hbm.at[uidx_v.at[sl]], ul_f.at[sl], sem)
            pltpu.async_copy(ilin_hbm.at[iidx_v.at[sl]], il_f.at[sl], sem)

        def srows(g, _):
            sl = pl.ds(g * L, L)
            usr_v[sl] = uidx_v[sl] >> 2
            isr_v[sl] = iidx_v[sl] >> 2
            return 0

        lax.fori_loop(0, BPW // L, srows, 0)

        def chunk(k):
            r0 = k * CH
            for c in range(CH // CK):
                isl = pl.ds(r0 + c * CK, CK)
                dsl = pl.ds(c * CK, CK)
                pltpu.async_copy(uemb_hbm.at[usr_v.at[isl]],
                                 ue_g.at[dsl], sem)
                pltpu.async_copy(iemb_hbm.at[isr_v.at[isl]],
                                 ie_g.at[dsl], sem)
            pltpu.make_async_copy(uemb_hbm.at[pl.ds(0, CH)], ue_g,
                                  sem).wait()
            pltpu.make_async_copy(iemb_hbm.at[pl.ds(0, CH)], ie_g,
                                  sem).wait()

            def ext(g, _):
                uv = uidx_v[pl.ds(r0 + g * L, L)] & 3
                iv = iidx_v[pl.ds(r0 + g * L, L)] & 3
                for j in range(L):
                    r2 = g * L + j
                    uq = uv[j] * D
                    iq = iv[j] * D
                    f = (r0 + r2) * D
                    ue_f[pl.ds(f, L)] = ue_g[r2, pl.ds(uq, L)]
                    ue_f[pl.ds(f + L, L)] = ue_g[r2, pl.ds(uq + L, L)]
                    ie_f[pl.ds(f, L)] = ie_g[r2, pl.ds(iq, L)]
                    ie_f[pl.ds(f + L, L)] = ie_g[r2, pl.ds(iq + L, L)]
                return 0

            lax.fori_loop(0, CH // L, ext, 0)

        for k in range(NCH):
            chunk(k)

        pltpu.make_async_copy(lin_out.at[pl.ds(0, BPW)], ul_f, sem).wait()
        pltpu.make_async_copy(lin_out.at[pl.ds(0, BPW)], il_f, sem).wait()

        def lsum(g, _):
            sl = pl.ds(g * L, L)
            lin_f[sl] = ul_f[sl] + il_f[sl]
            return 0

        lax.fori_loop(0, BPW // L, lsum, 0)

        pltpu.sync_copy(ue_f, ue_out.at[pl.ds(wid * FPW, FPW)])
        pltpu.sync_copy(ie_f, ie_out.at[pl.ds(wid * FPW, FPW)])
        pltpu.sync_copy(lin_f, lin_out.at[pl.ds(base, BPW)])

    return sc_gather


def _sc_gather(*args):
    return _make_sc_gather()(*args)


def _dense_body(ue_ref, ie_ref, lin_ref, w0u_ref, w0i_ref, b0_ref,
                w1_ref, b1_ref, wout_ref, ones_ref, c_ref, out_ref):
    ue = ue_ref[...]
    ie = ie_ref[...]
    dn = (((1,), (0,)), ((), ()))
    h0 = lax.dot_general(ue, w0u_ref[...], dn,
                         preferred_element_type=jnp.float32)
    h0 = h0 + lax.dot_general(ie, w0i_ref[...], dn,
                              preferred_element_type=jnp.float32)
    h0 = jnp.maximum(h0 + b0_ref[...], 0.0)
    h1 = lax.dot_general(h0, w1_ref[...], dn,
                         preferred_element_type=jnp.float32)
    h1 = jnp.maximum(h1 + b1_ref[...], 0.0)
    dnn = lax.dot_general(h1, wout_ref[...], dn,
                          preferred_element_type=jnp.float32)
    fm = lax.dot_general(ue * ie, ones_ref[...], dn,
                         preferred_element_type=jnp.float32)
    logit = lin_ref[0] + fm + dnn + c_ref[0]
    out_ref[...] = (1.0 / (1.0 + jnp.exp(-logit)))[None]


def _dense(ue, ie, lin, w0u, w0i, b0r, w1, b1r, wout, ones_blk, c):
    RB = 512
    grid = (BP // RB,)
    PD = PK * D
    return pl.pallas_call(
        _dense_body,
        grid=grid,
        in_specs=[
            pl.BlockSpec((RB, PD), lambda i: (i, 0)),
            pl.BlockSpec((RB, PD), lambda i: (i, 0)),
            pl.BlockSpec((1, RB, PK), lambda i: (i, 0, 0)),
            pl.BlockSpec((PD, PD), lambda i: (0, 0)),
            pl.BlockSpec((PD, PD), lambda i: (0, 0)),
            pl.BlockSpec((1, PD), lambda i: (0, 0)),
            pl.BlockSpec((PD, PD), lambda i: (0, 0)),
            pl.BlockSpec((1, PD), lambda i: (0, 0)),
            pl.BlockSpec((PD, PK), lambda i: (0, 0)),
            pl.BlockSpec((PD, PK), lambda i: (0, 0)),
            pl.BlockSpec(memory_space=pltpu.SMEM),
        ],
        out_specs=pl.BlockSpec((1, RB, PK), lambda i: (i, 0, 0)),
        out_shape=jax.ShapeDtypeStruct((BP // RB, RB, PK), jnp.float32),
    )(ue, ie, lin, w0u, w0i, b0r, w1, b1r, wout, ones_blk, c)


def kernel(user_ids, item_ids, user_embedding, item_embedding, user_linear,
           item_linear, W0, b0, g0, beta0, W1, b1, g1, beta1, W_out, b_out,
           bias):
    eps = 1e-5
    s = 1.0 / jnp.sqrt(1.0 + eps)
    s0 = g0 * s
    s1 = g1 * s
    W0f = W0 * s0[None, :]
    b0f = (b0 * s0 + beta0).reshape((1, D))
    W1f = W1 * s1[None, :]
    b1f = (b1 * s1 + beta1).reshape((1, D))
    c = (b_out + bias).reshape((1,))

    eye = jnp.eye(PK, dtype=jnp.float32)
    w0u_blk = jnp.kron(eye, W0f[:D])
    w0i_blk = jnp.kron(eye, W0f[D:])
    w1_blk = jnp.kron(eye, W1f)
    wout_blk = jnp.kron(eye, W_out)
    ones_blk = jnp.kron(eye, jnp.ones((D, 1), jnp.float32))
    b0t = jnp.tile(b0f, (1, PK))
    b1t = jnp.tile(b1f, (1, PK))

    uemb2 = user_embedding.reshape((NU // PK, PK * D))
    iemb2 = item_embedding.reshape((NU // PK, PK * D))
    ulin = user_linear.reshape((NU,))
    ilin = item_linear.reshape((NU,))

    uef, ief, lin = _sc_gather(
        user_ids.astype(jnp.int32), item_ids.astype(jnp.int32),
        uemb2, iemb2, ulin, ilin)
    ue2 = uef.reshape((BP, PK * D))
    ie2 = ief.reshape((BP, PK * D))
    lin2 = lin.reshape((BP // 512, 512, PK))

    out = _dense(ue2, ie2, lin2, w0u_blk, w0i_blk, b0t, w1_blk, b1t,
                 wout_blk, ones_blk, c)
    return out.reshape((B,))
